# baseline (speedup 1.0000x reference)
import jax
import jax.numpy as jnp
from jax.experimental import pallas as pl
from jax.experimental.pallas import tpu as pltpu

_E = 16
_K = 2
_BLK = 512


def _topk_part(lt):
    p = jax.nn.sigmoid(lt)
    iota = jax.lax.broadcasted_iota(jnp.int32, p.shape, 0)
    m1 = jnp.max(p, axis=0, keepdims=True)
    i1 = jnp.min(jnp.where(p == m1, iota, _E), axis=0, keepdims=True)
    pm = jnp.where(iota == i1, -1.0, p)
    m2 = jnp.max(pm, axis=0, keepdims=True)
    i2 = jnp.min(jnp.where(pm == m2, iota, _E), axis=0, keepdims=True)
    s = m1 + m2
    w1 = m1 / s
    w2 = m2 / s
    return (jnp.concatenate([w1, w2], axis=0),
            jnp.concatenate([i1, i2], axis=0),
            jnp.where(iota == i1, w1, jnp.where(iota == i2, w2, 0.0)))


def _router_block(xa_ref, xb_ref, w_ref, bt_ref,
                  pa_ref, ia_ref, ma_ref, pb_ref, ib_ref, mb_ref):
    w = w_ref[...]
    bt = bt_ref[...]
    dn = (((1,), (1,)), ((), ()))
    lta = jax.lax.dot_general(w, xa_ref[...], dimension_numbers=dn,
                              preferred_element_type=jnp.float32) + bt
    ltb = jax.lax.dot_general(w, xb_ref[...], dimension_numbers=dn,
                              preferred_element_type=jnp.float32) + bt
    pa_ref[...], ia_ref[...], ma_ref[...] = _topk_part(lta)
    pb_ref[...], ib_ref[...], mb_ref[...] = _topk_part(ltb)


def kernel(hidden_states, W, b):
    B, S, H = hidden_states.shape
    T = B * S
    Th = T // 2
    x = hidden_states.reshape(T, H)
    xa = x[:Th]
    xb = x[Th:]
    bt = b.reshape(_E, 1)
    grid = (Th // _BLK,)
    outs = pl.pallas_call(
        _router_block,
        grid=grid,
        in_specs=[
            pl.BlockSpec((_BLK, H), lambda i: (i, 0)),
            pl.BlockSpec((_BLK, H), lambda i: (i, 0)),
            pl.BlockSpec((_E, H), lambda i: (0, 0)),
            pl.BlockSpec((_E, 1), lambda i: (0, 0)),
        ],
        out_specs=[
            pl.BlockSpec((_K, _BLK), lambda i: (0, i)),
            pl.BlockSpec((_K, _BLK), lambda i: (0, i)),
            pl.BlockSpec((_E, _BLK), lambda i: (0, i)),
            pl.BlockSpec((_K, _BLK), lambda i: (0, i)),
            pl.BlockSpec((_K, _BLK), lambda i: (0, i)),
            pl.BlockSpec((_E, _BLK), lambda i: (0, i)),
        ],
        out_shape=[
            jax.ShapeDtypeStruct((_K, Th), jnp.float32),
            jax.ShapeDtypeStruct((_K, Th), jnp.int32),
            jax.ShapeDtypeStruct((_E, Th), jnp.float32),
            jax.ShapeDtypeStruct((_K, Th), jnp.float32),
            jax.ShapeDtypeStruct((_K, Th), jnp.int32),
            jax.ShapeDtypeStruct((_E, Th), jnp.float32),
        ],
        compiler_params=pltpu.CompilerParams(
            dimension_semantics=("parallel",)),
    )(xa, xb, W, bt)
    pa, ia, ma, pb, ib, mb = outs
    probs = jnp.concatenate([pa.T, pb.T], axis=0).reshape(B, S, _K)
    idx = jnp.concatenate([ia.T, ib.T], axis=0).reshape(B, S, _K)
    rmap = jnp.concatenate([ma.T, mb.T], axis=0).reshape(B, S, _E)
    return (probs, idx, rmap)


# speedup vs baseline: 2.5445x; 2.5445x over previous
import jax
import jax.numpy as jnp
from jax.experimental import pallas as pl
from jax.experimental.pallas import tpu as pltpu

_E = 16
_K = 2
_BLK = 512


def _topk_part(lt):
    p = jax.nn.sigmoid(lt)
    iota = jax.lax.broadcasted_iota(jnp.int32, p.shape, 0)
    m1 = jnp.max(p, axis=0, keepdims=True)
    i1 = jnp.min(jnp.where(p == m1, iota, _E), axis=0, keepdims=True)
    pm = jnp.where(iota == i1, -1.0, p)
    m2 = jnp.max(pm, axis=0, keepdims=True)
    i2 = jnp.min(jnp.where(pm == m2, iota, _E), axis=0, keepdims=True)
    s = m1 + m2
    w1 = m1 / s
    w2 = m2 / s
    return (jnp.concatenate([w1, w2], axis=0),
            jnp.concatenate([i1, i2], axis=0),
            jnp.where(iota == i1, w1, jnp.where(iota == i2, w2, 0.0)))


def _router_block(xa_ref, xb_ref, w_ref, bt_ref,
                  pa_ref, ia_ref, ma_ref, pb_ref, ib_ref, mb_ref):
    w = w_ref[...]
    bt = bt_ref[...]
    dn = (((1,), (1,)), ((), ()))
    lta = jax.lax.dot_general(w, xa_ref[...], dimension_numbers=dn,
                              preferred_element_type=jnp.float32) + bt
    ltb = jax.lax.dot_general(w, xb_ref[...], dimension_numbers=dn,
                              preferred_element_type=jnp.float32) + bt
    pa_ref[...], ia_ref[...], ma_ref[...] = _topk_part(lta)
    pb_ref[...], ib_ref[...], mb_ref[...] = _topk_part(ltb)


def kernel(hidden_states, W, b):
    B, S, H = hidden_states.shape
    T = B * S
    Th = T // 2
    x = hidden_states.reshape(T, H)
    bt = b.reshape(_E, 1)
    nblk = Th // _BLK
    grid = (nblk,)
    outs = pl.pallas_call(
        _router_block,
        grid=grid,
        in_specs=[
            pl.BlockSpec((_BLK, H), lambda i: (i, 0)),
            pl.BlockSpec((_BLK, H), lambda i: (i + nblk, 0)),
            pl.BlockSpec((_E, H), lambda i: (0, 0)),
            pl.BlockSpec((_E, 1), lambda i: (0, 0)),
        ],
        out_specs=[
            pl.BlockSpec((_K, _BLK), lambda i: (0, i)),
            pl.BlockSpec((_K, _BLK), lambda i: (0, i)),
            pl.BlockSpec((_E, _BLK), lambda i: (0, i)),
            pl.BlockSpec((_K, _BLK), lambda i: (0, i)),
            pl.BlockSpec((_K, _BLK), lambda i: (0, i)),
            pl.BlockSpec((_E, _BLK), lambda i: (0, i)),
        ],
        out_shape=[
            jax.ShapeDtypeStruct((_K, Th), jnp.float32),
            jax.ShapeDtypeStruct((_K, Th), jnp.int32),
            jax.ShapeDtypeStruct((_E, Th), jnp.float32),
            jax.ShapeDtypeStruct((_K, Th), jnp.float32),
            jax.ShapeDtypeStruct((_K, Th), jnp.int32),
            jax.ShapeDtypeStruct((_E, Th), jnp.float32),
        ],
        compiler_params=pltpu.CompilerParams(
            dimension_semantics=("parallel",)),
    )(x, x, W, bt)
    pa, ia, ma, pb, ib, mb = outs
    probs = jnp.concatenate([pa.T, pb.T], axis=0).reshape(B, S, _K)
    idx = jnp.concatenate([ia.T, ib.T], axis=0).reshape(B, S, _K)
    rmap = jnp.concatenate([ma.T, mb.T], axis=0).reshape(B, S, _E)
    return (probs, idx, rmap)


# speedup vs baseline: 2.8433x; 1.1174x over previous
import jax
import jax.numpy as jnp
from jax.experimental import pallas as pl
from jax.experimental.pallas import tpu as pltpu

_E = 16
_K = 2
_BLK = 1024


def _router_block(x_ref, w_ref, bt_ref, p_ref, i_ref, m_ref):
    lt = jax.lax.dot_general(
        w_ref[...], x_ref[...],
        dimension_numbers=(((1,), (1,)), ((), ())),
        preferred_element_type=jnp.float32) + bt_ref[...]
    p = jax.nn.sigmoid(lt)
    iota = jax.lax.broadcasted_iota(jnp.int32, p.shape, 0)
    m1 = jnp.max(p, axis=0, keepdims=True)
    i1 = jnp.min(jnp.where(p == m1, iota, _E), axis=0, keepdims=True)
    pm = jnp.where(iota == i1, -1.0, p)
    m2 = jnp.max(pm, axis=0, keepdims=True)
    i2 = jnp.min(jnp.where(pm == m2, iota, _E), axis=0, keepdims=True)
    s = m1 + m2
    w1 = m1 / s
    w2 = m2 / s
    p_ref[...] = jnp.concatenate([w1, w2], axis=0)
    i_ref[...] = jnp.concatenate([i1, i2], axis=0)
    m_ref[...] = jnp.where(iota == i1, w1, jnp.where(iota == i2, w2, 0.0))


def kernel(hidden_states, W, b):
    B, S, H = hidden_states.shape
    T = B * S
    x = hidden_states.reshape(T, H)
    bt = b.reshape(_E, 1)
    grid = (T // _BLK,)
    probs_t, idx_t, rmap_t = pl.pallas_call(
        _router_block,
        grid=grid,
        in_specs=[
            pl.BlockSpec((_BLK, H), lambda i: (i, 0)),
            pl.BlockSpec((_E, H), lambda i: (0, 0)),
            pl.BlockSpec((_E, 1), lambda i: (0, 0)),
        ],
        out_specs=[
            pl.BlockSpec((_K, _BLK), lambda i: (0, i)),
            pl.BlockSpec((_K, _BLK), lambda i: (0, i)),
            pl.BlockSpec((_E, _BLK), lambda i: (0, i)),
        ],
        out_shape=[
            jax.ShapeDtypeStruct((_K, T), jnp.float32),
            jax.ShapeDtypeStruct((_K, T), jnp.int32),
            jax.ShapeDtypeStruct((_E, T), jnp.float32),
        ],
        compiler_params=pltpu.CompilerParams(
            dimension_semantics=("arbitrary",)),
    )(x, W, bt)
    return (probs_t.T.reshape(B, S, _K), idx_t.T.reshape(B, S, _K),
            rmap_t.T.reshape(B, S, _E))


# speedup vs baseline: 2.8444x; 1.0004x over previous
import jax
import jax.numpy as jnp
from jax.experimental import pallas as pl
from jax.experimental.pallas import tpu as pltpu

_E = 16
_K = 2
_BLK = 1024


def _router_block(x_ref, w_ref, bt_ref, p_ref, i_ref, m_ref):
    lt = jax.lax.dot_general(
        w_ref[...], x_ref[...],
        dimension_numbers=(((1,), (1,)), ((), ())),
        preferred_element_type=jnp.float32) + bt_ref[...]
    p = jax.nn.sigmoid(lt)
    iota = jax.lax.broadcasted_iota(jnp.int32, p.shape, 0)
    m1 = jnp.max(p, axis=0, keepdims=True)
    i1 = jnp.min(jnp.where(p == m1, iota, _E), axis=0, keepdims=True)
    pm = jnp.where(iota == i1, -1.0, p)
    m2 = jnp.max(pm, axis=0, keepdims=True)
    i2 = jnp.min(jnp.where(pm == m2, iota, _E), axis=0, keepdims=True)
    s = m1 + m2
    w1 = m1 / s
    w2 = m2 / s
    p_ref[...] = jnp.concatenate([w1, w2], axis=0)
    i_ref[...] = jnp.concatenate([i1, i2], axis=0)
    m_ref[...] = jnp.where(iota == i1, w1, jnp.where(iota == i2, w2, 0.0))


def kernel(hidden_states, W, b):
    B, S, H = hidden_states.shape
    T = B * S
    x = hidden_states.reshape(T, H)
    bt = b.reshape(_E, 1)
    grid = (T // _BLK,)
    probs_t, idx_t, rmap_t = pl.pallas_call(
        _router_block,
        grid=grid,
        in_specs=[
            pl.BlockSpec((_BLK, H), lambda i: (i, 0)),
            pl.BlockSpec((_E, H), lambda i: (0, 0)),
            pl.BlockSpec((_E, 1), lambda i: (0, 0)),
        ],
        out_specs=[
            pl.BlockSpec((_K, _BLK), lambda i: (0, i)),
            pl.BlockSpec((_K, _BLK), lambda i: (0, i)),
            pl.BlockSpec((_E, _BLK), lambda i: (0, i)),
        ],
        out_shape=[
            jax.ShapeDtypeStruct((_K, T), jnp.float32),
            jax.ShapeDtypeStruct((_K, T), jnp.int32),
            jax.ShapeDtypeStruct((_E, T), jnp.float32),
        ],
        compiler_params=pltpu.CompilerParams(
            dimension_semantics=("parallel",)),
    )(x, W, bt)
    return (probs_t.T.reshape(B, S, _K), idx_t.T.reshape(B, S, _K),
            rmap_t.T.reshape(B, S, _E))
